# Initial kernel scaffold; baseline (speedup 1.0000x reference)
import jax
import jax.numpy as jnp
from jax.experimental import pallas as pl


def kernel(data2):
    raise NotImplementedError("write your pallas kernel here")


# speedup vs baseline: 4.3403x; 4.3403x over previous
import jax
import jax.numpy as jnp
from jax.experimental import pallas as pl

_IDXC = (0, 0, 1, 1, 2, 2, 3, 3, 4, 4, 5, 5, 5, 6, 6, 7, 7, 8, 8, 9, 9)
_D = 64


def _body(x_ref, o_ref):
    x = x_ref[...]
    for j, i in enumerate(_IDXC):
        o_ref[:, :, j * _D:(j + 1) * _D] = x[:, :, i * _D:(i + 1) * _D]


def kernel(data2):
    B, F, N, D = data2.shape
    x = data2.reshape(B, F, N * D)
    B_blk = 128
    out = pl.pallas_call(
        _body,
        grid=(B // B_blk,),
        in_specs=[pl.BlockSpec((B_blk, F, N * D), lambda b: (b, 0, 0))],
        out_specs=pl.BlockSpec((B_blk, F, len(_IDXC) * D), lambda b: (b, 0, 0)),
        out_shape=jax.ShapeDtypeStruct((B, F, len(_IDXC) * D), jnp.float32),
    )(x)
    return out.reshape(B, F, len(_IDXC), D)
